# baseline (speedup 1.0000x reference)
import functools
import math

import jax
import jax.numpy as jnp
from jax import lax
from jax.experimental import pallas as pl
from jax.experimental.pallas import tpu as pltpu
from jax.experimental.pallas import tpu_sc as plsc

D_MODEL = 64
SCALE = math.sqrt(D_MODEL)
LANES = 16


@functools.lru_cache(maxsize=None)
def _build_call(batch: int, vocab: int, d: int, chunk: int):
    info = plsc.get_sparse_core_info()
    nc, ns = info.num_cores, info.num_subcores
    nw = nc * ns
    assert batch % (nw * chunk) == 0
    b_per_w = batch // nw
    iters = b_per_w // chunk
    mesh = plsc.VectorSubcoreMesh(core_axis_name="c", subcore_axis_name="s")

    @functools.partial(
        pl.kernel,
        mesh=mesh,
        out_type=jax.ShapeDtypeStruct((batch, d), jnp.float32),
        scratch_types=[
            pltpu.VMEM((chunk,), jnp.int32),
            pltpu.VMEM((chunk, d), jnp.float32),
            pltpu.SemaphoreType.DMA,
        ],
        compiler_params=pltpu.CompilerParams(use_tc_tiling_on_sc=False),
    )
    def emb(table_hbm, idx_hbm, out_hbm, idx_v, rows_v, sem):
        wid = lax.axis_index("s") * nc + lax.axis_index("c")
        base = wid * b_per_w

        def chunk_body(g, carry):
            off = base + g * chunk
            pltpu.sync_copy(idx_hbm.at[pl.ds(off, chunk)], idx_v)
            pltpu.async_copy(table_hbm.at[idx_v], rows_v, sem).wait()

            def mul_body(i, c):
                for j in range(d // LANES):
                    sl = pl.ds(j * LANES, LANES)
                    rows_v[i, sl] = rows_v[i, sl] * SCALE
                return c

            lax.fori_loop(0, chunk, mul_body, 0)
            pltpu.sync_copy(rows_v, out_hbm.at[pl.ds(off, chunk)])
            return carry

        lax.fori_loop(0, iters, chunk_body, 0)

    return emb


def kernel(x, table):
    vocab, d = table.shape
    batch = x.shape[0] * x.shape[1]
    idx = x.reshape(-1).astype(jnp.int32)
    out = _build_call(batch, vocab, d, 512)(table, idx)
    return out.reshape(x.shape + (d,))


# speedup vs baseline: 1.1368x; 1.1368x over previous
import functools
import math

import jax
import jax.numpy as jnp
from jax import lax
from jax.experimental import pallas as pl
from jax.experimental.pallas import tpu as pltpu
from jax.experimental.pallas import tpu_sc as plsc

D_MODEL = 64
SCALE = math.sqrt(D_MODEL)
LANES = 16
CHUNK = 320
NBUF = 4


@functools.lru_cache(maxsize=None)
def _build_call(batch: int, vocab: int, d: int):
    info = plsc.get_sparse_core_info()
    nc, ns = info.num_cores, info.num_subcores
    nw = nc * ns
    assert batch % (nw * CHUNK * NBUF) == 0
    b_per_w = batch // nw
    iters = b_per_w // CHUNK
    mesh = plsc.VectorSubcoreMesh(core_axis_name="c", subcore_axis_name="s")

    @functools.partial(
        pl.kernel,
        mesh=mesh,
        out_type=jax.ShapeDtypeStruct((batch, d), jnp.float32),
        scratch_types=[
            pltpu.VMEM((b_per_w,), jnp.int32),
            pltpu.VMEM((NBUF, CHUNK, d), jnp.float32),
        ]
        + [pltpu.SemaphoreType.DMA] * (2 * NBUF),
        compiler_params=pltpu.CompilerParams(use_tc_tiling_on_sc=False),
    )
    def emb(table_hbm, idx_hbm, out_hbm, idx_v, rows_v, *sems):
        sg, sw = sems[:NBUF], sems[NBUF:]
        wid = lax.axis_index("s") * nc + lax.axis_index("c")
        base = wid * b_per_w
        pltpu.sync_copy(idx_hbm.at[pl.ds(base, b_per_w)], idx_v)

        def gather_start(g, b):
            pltpu.async_copy(
                table_hbm.at[idx_v.at[pl.ds(g * CHUNK, CHUNK)]],
                rows_v.at[b],
                sg[b],
            )

        for b in range(NBUF - 1):
            gather_start(b, b)

        @pl.loop(0, iters, step=NBUF)
        def outer(gg):
            for b in range(NBUF):
                g = gg + b
                out_slice = out_hbm.at[pl.ds(base + g * CHUNK, CHUNK)]
                pltpu.make_async_copy(
                    table_hbm.at[idx_v.at[pl.ds(0, CHUNK)]], rows_v.at[b], sg[b]
                ).wait()

                @plsc.parallel_loop(0, CHUNK, unroll=4)
                def mul(i):
                    for j in range(d // LANES):
                        sl = pl.ds(j * LANES, LANES)
                        rows_v[b, i, sl] = rows_v[b, i, sl] * SCALE

                pltpu.async_copy(rows_v.at[b], out_slice, sw[b])

                nxt = g + NBUF - 1
                bf = (b + NBUF - 1) % NBUF

                @pl.when(jnp.logical_and(nxt < iters, g >= 1))
                def _():
                    pltpu.make_async_copy(
                        rows_v.at[bf],
                        out_hbm.at[pl.ds(base, CHUNK)],
                        sw[bf],
                    ).wait()
                    gather_start(nxt, bf)

                @pl.when(jnp.logical_and(nxt < iters, g < 1))
                def _():
                    gather_start(nxt, bf)

        for b in range(NBUF):
            pltpu.make_async_copy(
                rows_v.at[b], out_hbm.at[pl.ds(base, CHUNK)], sw[b]
            ).wait()

    return emb


def kernel(x, table):
    vocab, d = table.shape
    batch = x.shape[0] * x.shape[1]
    idx = x.reshape(-1).astype(jnp.int32)
    out = _build_call(batch, vocab, d)(table, idx)
    return out.reshape(x.shape + (d,))
